# baseline (speedup 1.0000x reference)
import functools

import jax
import jax.numpy as jnp
from jax import lax
from jax.experimental import pallas as pl
from jax.experimental.pallas import tpu as pltpu
from jax.experimental.pallas import tpu_sc as plsc

B = 16384
H = 20
D = 32
V = 1000000
NW = 32
BPW = B // NW
TILE_B = 64
NT = BPW // TILE_B

CPB = 512
NCOL = (V + CPB - 1) // CPB
VP = NCOL * CPB
VFULL = (V // CPB) * CPB
NPAIR = (NCOL // NW + 2) // 2


@functools.partial(
    pl.kernel,
    mesh=plsc.VectorSubcoreMesh(core_axis_name="c", subcore_axis_name="s"),
    out_type=jax.ShapeDtypeStruct((VP * D // 128, 128), jnp.float32),
    compiler_params=pltpu.CompilerParams(needs_layout_passes=False),
    scratch_types=[
        pltpu.VMEM((D, CPB), jnp.float32),
        pltpu.VMEM((D, CPB), jnp.float32),
        pltpu.VMEM((CPB * D // 128, 128), jnp.float32),
        pltpu.VMEM((CPB * D // 128, 128), jnp.float32),
        pltpu.SemaphoreType.DMA,
        pltpu.SemaphoreType.DMA,
        pltpu.SemaphoreType.DMA,
        pltpu.SemaphoreType.DMA,
    ],
)
def _repack(tblt_hbm, tail_hbm, out_hbm,
            in_a, in_b, out_a, out_b, isem_a, isem_b, osem_a, osem_b):
    wid = lax.axis_index("s") * 2 + lax.axis_index("c")
    lane = lax.iota(jnp.int32, 16)
    lane_hi = lane + 16
    zeros16 = jnp.zeros((16,), jnp.int32)

    ins = ((in_a, isem_a), (in_b, isem_b))
    outs = ((out_a, osem_a), (out_b, osem_b))

    def in_descr(c, par):
        buf, sem = ins[par]
        full = [
            pltpu.make_async_copy(
                tblt_hbm.at[pl.ds(g * 8, 8), pl.ds(c * CPB, CPB)],
                buf.at[pl.ds(g * 8, 8)], sem)
            for g in range(D // 8)
        ]
        part = pltpu.make_async_copy(tail_hbm, buf, sem)
        return full, part

    def fire_in(c, par):
        full, part = in_descr(c, par)

        @pl.when(c < NCOL - 1)
        def _():
            for f in full:
                f.start()

        @pl.when(c == NCOL - 1)
        def _():
            part.start()

    def wait_in(c, par):
        full, part = in_descr(c, par)

        @pl.when(c < NCOL - 1)
        def _():
            for f in full:
                f.wait()

        @pl.when(c == NCOL - 1)
        def _():
            part.wait()

    OROWS = CPB * D // 128

    def out_descr(c, par):
        buf, sem = outs[par]
        return pltpu.make_async_copy(
            buf, out_hbm.at[pl.ds(c * OROWS, OROWS)], sem)

    def fire_out(c, par):
        @pl.when(c < NCOL)
        def _():
            out_descr(c, par).start()

    def wait_out(c, par):
        @pl.when(c < NCOL)
        def _():
            out_descr(c, par).wait()

    def col_of(i):
        return wid + i * NW

    @pl.when(col_of(0) < NCOL)
    def _():
        fire_in(col_of(0), 0)

    def pair(j, _):
        for par in (0, 1):
            i = j * 2 + par
            c = col_of(i)

            @pl.when(c < NCOL)
            def _(i=i, c=c, par=par):
                cn = col_of(i + 1)

                @pl.when(cn < NCOL)
                def _():
                    fire_in(cn, 1 - par)

                wait_in(c, par)

                @pl.when(i >= 2)
                def _():
                    wait_out(col_of(i - 2), par)

                ibuf = ins[par][0]
                obuf = outs[par][0]

                def tr(r, _):
                    base = r * 4
                    for k in range(4):
                        cv = zeros16 + (base + k)
                        g0 = plsc.load_gather(ibuf, [lane, cv])
                        g1 = plsc.load_gather(ibuf, [lane_hi, cv])
                        obuf[r, pl.ds(k * D, 16)] = g0
                        obuf[r, pl.ds(k * D + 16, 16)] = g1
                    return 0

                lax.fori_loop(0, CPB * D // 128, tr, 0)
                fire_out(c, par)
        return 0

    lax.fori_loop(0, NPAIR, pair, 0)

    last = NPAIR * 2 - 1
    for i in (last - 3, last - 2, last - 1, last):
        @pl.when(jnp.logical_and(col_of(i) < NCOL, col_of(i + 2) >= NCOL))
        def _(i=i):
            wait_out(col_of(i), i % 2)


@functools.partial(
    pl.kernel,
    mesh=plsc.VectorSubcoreMesh(core_axis_name="c", subcore_axis_name="s"),
    out_type=jax.ShapeDtypeStruct((D, B), jnp.float32),
    compiler_params=pltpu.CompilerParams(
        use_tc_tiling_on_sc=False, needs_layout_passes=False),
    scratch_types=[
        pltpu.VMEM((H, BPW), jnp.int32),
        pltpu.VMEM((H * TILE_B, D), jnp.float32),
        pltpu.VMEM((H * TILE_B, D), jnp.float32),
        pltpu.VMEM((D, BPW), jnp.float32),
        pltpu.SemaphoreType.DMA,
        pltpu.SemaphoreType.DMA,
    ],
)
def _emb_lookup(idx_hbm, table_hbm, out_hbm,
                idx_v, rows_a, rows_b, out_v, sem_a, sem_b):
    wid = lax.axis_index("s") * 2 + lax.axis_index("c")
    pltpu.sync_copy(idx_hbm.at[:, pl.ds(wid * BPW, BPW)], idx_v)

    lane = lax.iota(jnp.int32, 16)
    row_hi = lane + 16
    zeros16 = jnp.zeros((16,), jnp.int32)
    bufs = ((rows_a, sem_a), (rows_b, sem_b))

    def fire(t):
        buf, sem = bufs[t % 2]
        return [
            pltpu.async_copy(
                table_hbm.at[idx_v.at[h, pl.ds(t * TILE_B, TILE_B)]],
                buf.at[pl.ds(h * TILE_B, TILE_B)],
                sem,
            )
            for h in range(H)
        ]

    inflight = fire(0)
    for t in range(NT):
        nxt = fire(t + 1) if t + 1 < NT else []
        for cp in inflight:
            cp.wait()
        inflight = nxt
        buf, _ = bufs[t % 2]

        def body(b, _, buf=buf, t=t):
            acc0 = buf[b, pl.ds(0, 16)]
            acc1 = buf[b, pl.ds(16, 16)]
            for h in range(1, H):
                acc0 = acc0 + buf[h * TILE_B + b, pl.ds(0, 16)]
                acc1 = acc1 + buf[h * TILE_B + b, pl.ds(16, 16)]
            col = zeros16 + (t * TILE_B + b)
            plsc.store_scatter(out_v, [lane, col], acc0 * (1.0 / H))
            plsc.store_scatter(out_v, [row_hi, col], acc1 * (1.0 / H))
            return 0

        lax.fori_loop(0, TILE_B, body, 0)

    pltpu.sync_copy(out_v, out_hbm.at[:, pl.ds(wid * BPW, BPW)])


def kernel(inputs, table):
    tail_t = jnp.pad(table[VFULL:], ((0, CPB - (V - VFULL)), (0, 0))).T
    packed = _repack(table.T, tail_t)
    table_rm = packed.reshape(VP, D)
    out_t = _emb_lookup(inputs.astype(jnp.int32).T, table_rm)
    return out_t.T


# speedup vs baseline: 1.3547x; 1.3547x over previous
import functools

import jax
import jax.numpy as jnp
from jax import lax
from jax.experimental import pallas as pl
from jax.experimental.pallas import tpu as pltpu
from jax.experimental.pallas import tpu_sc as plsc

B = 16384
H = 20
D = 32
NW = 32
BPW = B // NW
TILE_B = 64
NT = BPW // TILE_B


@functools.partial(
    pl.kernel,
    mesh=plsc.VectorSubcoreMesh(core_axis_name="c", subcore_axis_name="s"),
    out_type=jax.ShapeDtypeStruct((D, B), jnp.float32),
    compiler_params=pltpu.CompilerParams(
        use_tc_tiling_on_sc=False, needs_layout_passes=False),
    scratch_types=[
        pltpu.VMEM((H, BPW), jnp.int32),
        pltpu.VMEM((H * TILE_B, D), jnp.float32),
        pltpu.VMEM((H * TILE_B, D), jnp.float32),
        pltpu.VMEM((D, BPW), jnp.float32),
        pltpu.SemaphoreType.DMA,
        pltpu.SemaphoreType.DMA,
    ],
)
def _emb_lookup(idx_hbm, table_hbm, out_hbm,
                idx_v, rows_a, rows_b, out_v, sem_a, sem_b):
    wid = lax.axis_index("s") * 2 + lax.axis_index("c")
    pltpu.sync_copy(idx_hbm.at[:, pl.ds(wid * BPW, BPW)], idx_v)

    lane = lax.iota(jnp.int32, 16)
    row_hi = lane + 16
    zeros16 = jnp.zeros((16,), jnp.int32)
    bufs = ((rows_a, sem_a), (rows_b, sem_b))

    def fire(t):
        buf, sem = bufs[t % 2]
        return [
            pltpu.async_copy(
                table_hbm.at[idx_v.at[h, pl.ds(t * TILE_B, TILE_B)]],
                buf.at[pl.ds(h * TILE_B, TILE_B)],
                sem,
            )
            for h in range(H)
        ]

    inflight = fire(0)
    for t in range(NT):
        nxt = fire(t + 1) if t + 1 < NT else []
        for cp in inflight:
            cp.wait()
        inflight = nxt
        buf, _ = bufs[t % 2]

        def body(b, _, buf=buf, t=t):
            acc0 = buf[b, pl.ds(0, 16)]
            acc1 = buf[b, pl.ds(16, 16)]
            for h in range(1, H):
                acc0 = acc0 + buf[h * TILE_B + b, pl.ds(0, 16)]
                acc1 = acc1 + buf[h * TILE_B + b, pl.ds(16, 16)]
            col = zeros16 + (t * TILE_B + b)
            plsc.store_scatter(out_v, [lane, col], acc0 * (1.0 / H))
            plsc.store_scatter(out_v, [row_hi, col], acc1 * (1.0 / H))
            return 0

        lax.fori_loop(0, TILE_B, body, 0)

    pltpu.sync_copy(out_v, out_hbm.at[:, pl.ds(wid * BPW, BPW)])


def kernel(inputs, table):
    out_t = _emb_lookup(inputs.astype(jnp.int32).T, table)
    return out_t.T
